# Initial kernel scaffold; baseline (speedup 1.0000x reference)
import jax
import jax.numpy as jnp
from jax.experimental import pallas as pl


def kernel(adjacency_matrices_list, W1, b1, W2, b2, A1, ab1, A2, ab2, M1, mb1, M2, mb2):
    raise NotImplementedError("write your pallas kernel here")


# speedup vs baseline: 4674.8361x; 4674.8361x over previous
import jax
import jax.numpy as jnp
from jax.experimental import pallas as pl
from jax.experimental.pallas import tpu as pltpu

N = 1024
V = 3
H = 128
ATT = 64
OUT = 128


def _body(a_ref, W1_ref, b1_ref, W2_ref, b2_ref,
          A1_ref, ab1_ref, A2_ref, ab2_ref, M1_ref, mb1_ref, M2_ref, mb2_ref,
          fused_ref, weights_ref, stacked_ref, summ_ref):
    v = pl.program_id(0)

    a = a_ref[0]
    rows = jax.lax.broadcasted_iota(jnp.int32, (N, N), 0)
    cols = jax.lax.broadcasted_iota(jnp.int32, (N, N), 1)
    eye = rows == cols
    m = jnp.where((a != 0.0) | eye, 1.0, 0.0)
    deg = jnp.sum(m, axis=0)
    dinv = jax.lax.rsqrt(deg)

    b1 = b1_ref[v]
    b2 = b2_ref[v]

    dh = dinv[:, None] * W1_ref[0]
    t = jax.lax.dot_general(m, dh, (((0,), (0,)), ((), ())),
                            preferred_element_type=jnp.float32)
    h1 = jax.nn.relu(dinv[:, None] * t + b1[None, :])

    g = jnp.dot(h1, W2_ref[0], preferred_element_type=jnp.float32)
    dg = dinv[:, None] * g
    t2 = jax.lax.dot_general(m, dg, (((0,), (0,)), ((), ())),
                             preferred_element_type=jnp.float32)
    h2 = jax.nn.relu(dinv[:, None] * t2 + b2[None, :])

    stacked_ref[v] = h2
    summ_ref[pl.ds(v, 1), :] = jnp.mean(h2, axis=0, keepdims=True)

    @pl.when(v == V - 1)
    def _fusion():
        summ = summ_ref[...]
        t1 = jnp.tanh(jnp.dot(summ, A1_ref[...],
                              preferred_element_type=jnp.float32)
                      + ab1_ref[...][None, :])
        s = jnp.dot(t1, A2_ref[...],
                    preferred_element_type=jnp.float32) + ab2_ref[...][None, :]
        s = s - jnp.max(s, axis=0, keepdims=True)
        e = jnp.exp(s)
        w = e / jnp.sum(e, axis=0, keepdims=True)
        weights_ref[...] = w

        st = stacked_ref[...]
        fusion = jnp.concatenate(
            [w[i, 0] * st[i] for i in range(V)], axis=1)
        hidden = jax.nn.relu(
            jnp.dot(fusion, M1_ref[...], preferred_element_type=jnp.float32)
            + mb1_ref[...][None, :])
        fused_ref[...] = (jnp.dot(hidden, M2_ref[...],
                                  preferred_element_type=jnp.float32)
                          + mb2_ref[...][None, :])


def kernel(adjacency_matrices_list, W1, b1, W2, b2, A1, ab1, A2, ab2,
           M1, mb1, M2, mb2):
    grid = (V,)
    full = lambda shape: pl.BlockSpec(shape, lambda v: tuple(0 for _ in shape))
    in_specs = [
        pl.BlockSpec((1, N, N), lambda v: (v, 0, 0)),
        pl.BlockSpec((1, N, H), lambda v: (v, 0, 0)),
        full((V, H)),
        pl.BlockSpec((1, H, H), lambda v: (v, 0, 0)),
        full((V, H)),
        full((H, ATT)), full((ATT,)), full((ATT, 1)), full((1,)),
        full((V * H, 2 * H)), full((2 * H,)), full((2 * H, OUT)), full((OUT,)),
    ]
    out_specs = [
        full((N, OUT)),
        full((V, 1)),
        full((V, N, H)),
    ]
    out_shapes = [
        jax.ShapeDtypeStruct((N, OUT), jnp.float32),
        jax.ShapeDtypeStruct((V, 1), jnp.float32),
        jax.ShapeDtypeStruct((V, N, H), jnp.float32),
    ]
    fused, w, stacked = pl.pallas_call(
        _body,
        grid=grid,
        in_specs=in_specs,
        out_specs=out_specs,
        out_shape=out_shapes,
        scratch_shapes=[pltpu.VMEM((V, H), jnp.float32)],
    )(adjacency_matrices_list, W1, b1, W2, b2, A1, ab1, A2, ab2,
      M1, mb1, M2, mb2)
    return fused, w[:, 0], stacked
